# Initial kernel scaffold; baseline (speedup 1.0000x reference)
import jax
import jax.numpy as jnp
from jax.experimental import pallas as pl


def kernel(context, targets, W, b):
    raise NotImplementedError("write your pallas kernel here")


# speedup vs baseline: 12.1216x; 12.1216x over previous
import functools

import jax
import jax.numpy as jnp
from jax import lax
from jax.experimental import pallas as pl
from jax.experimental.pallas import tpu as pltpu
from jax.experimental.pallas import tpu_sc as plsc

_B, _T, _D = 4, 512, 256
_P, _K = 12, 100
_INVTEMP = 10.0
_NR = _B * _T
_NROWS = _P * _NR
_LANES = 16
_KTOT = _K + 1
_NG = _NROWS // _LANES
_NCORES, _NSUB = 2, 16
_NW = _NCORES * _NSUB
_GPW = _NG // _NW


def _build_gather_cols():
    r_in_step = jnp.arange(_NR, dtype=jnp.int32)
    cols = []
    for s in range(_P):
        step = s + 1
        t2 = _T - step
        kstep = jax.random.fold_in(jax.random.key(42), step)
        neg = jax.random.randint(kstep, (_B, t2, _K), 0, _B * t2)
        negc = (neg // t2) * _T + (neg % t2) + step
        negc = jnp.pad(negc, ((0, 0), (0, _T - t2), (0, 0)))
        posc = jnp.minimum(r_in_step + step, _NR - 1).reshape(_B, _T, 1)
        cols.append(jnp.concatenate([posc, negc], axis=2))
    allc = jnp.stack(cols).reshape(_NG, _LANES, _KTOT)
    allc = allc + (jnp.arange(_LANES, dtype=jnp.int32) * _NR)[None, :, None]
    return allc.transpose(0, 2, 1).astype(jnp.int32)


def _sim_body(ctx_ref, tgt_ref, w_ref, b_ref, s_ref):
    tgt = tgt_ref[...]
    tn = tgt / jnp.maximum(
        jnp.sqrt(jnp.sum(tgt * tgt, axis=1, keepdims=True)), 1e-12)
    q = lax.dot_general(ctx_ref[...], w_ref[0], (((1,), (1,)), ((), ())),
                        preferred_element_type=jnp.float32)
    q = q + b_ref[0, 0][None, :]
    qn = q / jnp.maximum(
        jnp.sqrt(jnp.sum(q * q, axis=1, keepdims=True)), 1e-12)
    s_ref[...] = lax.dot_general(qn, tn, (((1,), (1,)), ((), ())),
                                 preferred_element_type=jnp.float32)


_sim = pl.pallas_call(
    _sim_body,
    grid=(_P,),
    in_specs=[
        pl.BlockSpec((_NR, _D), lambda s: (0, 0)),
        pl.BlockSpec((_NR, _D), lambda s: (0, 0)),
        pl.BlockSpec((1, _D, _D), lambda s: (s, 0, 0)),
        pl.BlockSpec((1, 1, _D), lambda s: (s, 0, 0)),
    ],
    out_specs=pl.BlockSpec((_NR, _NR), lambda s: (s, 0)),
    out_shape=jax.ShapeDtypeStruct((_NROWS, _NR), jnp.float32),
)


def _sc_gather_body(s_hbm, idx_hbm, sum_hbm, pos_hbm,
                    rowbuf, idxbuf, vsum, vpos):
    cid = lax.axis_index("c")
    sid = lax.axis_index("s")
    wid = sid * _NCORES + cid

    def body(g, carry):
        gg = wid * _GPW + g
        grow = gg * _LANES
        pltpu.sync_copy(s_hbm.at[pl.ds(grow * _NR, _LANES * _NR)], rowbuf)
        pltpu.sync_copy(idx_hbm.at[gg], idxbuf)
        v = plsc.load_gather(rowbuf, [idxbuf[0, :]])
        vpos[:] = v
        acc = jnp.exp(v * _INVTEMP)
        for k in range(1, _KTOT):
            vk = plsc.load_gather(rowbuf, [idxbuf[k, :]])
            acc = acc + jnp.exp(vk * _INVTEMP)
        vsum[:] = acc
        pltpu.sync_copy(vsum, sum_hbm.at[pl.ds(grow, _LANES)])
        pltpu.sync_copy(vpos, pos_hbm.at[pl.ds(grow, _LANES)])
        return carry

    lax.fori_loop(0, _GPW, body, 0)


@functools.cache
def _get_sc_gather():
    return pl.kernel(
        _sc_gather_body,
        mesh=plsc.VectorSubcoreMesh(core_axis_name="c", subcore_axis_name="s"),
        compiler_params=pltpu.CompilerParams(needs_layout_passes=False),
        out_type=[jax.ShapeDtypeStruct((_NROWS,), jnp.float32),
                  jax.ShapeDtypeStruct((_NROWS,), jnp.float32)],
        scratch_types=[
            pltpu.VMEM((_LANES * _NR,), jnp.float32),
            pltpu.VMEM((_KTOT, _LANES), jnp.int32),
            pltpu.VMEM((_LANES,), jnp.float32),
            pltpu.VMEM((_LANES,), jnp.float32),
        ],
    )


def _loss_body(sum_ref, pos_ref, out_ref):
    se = sum_ref[...]
    ps = pos_ref[...]
    srow = lax.broadcasted_iota(jnp.int32, (_P, _NR), 0)
    rcol = lax.broadcasted_iota(jnp.int32, (_P, _NR), 1)
    t2 = (_T - 1) - srow
    valid = (rcol % _T) < t2
    wgt = jnp.where(valid, 1.0, 0.0) / (_P * _B * t2.astype(jnp.float32))
    out_ref[...] = jnp.sum((jnp.log(se) - ps * _INVTEMP) * wgt).reshape(1, 1)


_loss = pl.pallas_call(
    _loss_body,
    out_shape=jax.ShapeDtypeStruct((1, 1), jnp.float32),
)


def kernel(context, targets, W, b):
    ctx2 = context.reshape(_NR, _D)
    tgt2 = targets.reshape(_NR, _D)
    b3 = b.reshape(_P, 1, _D)
    idx = _build_gather_cols()
    sim = _sim(ctx2, tgt2, W, b3)
    se, ps = _get_sc_gather()(sim.reshape(-1), idx)
    return _loss(se.reshape(_P, _NR), ps.reshape(_P, _NR)).reshape(1)


# speedup vs baseline: 16.1922x; 1.3358x over previous
import functools

import jax
import jax.numpy as jnp
from jax import lax
from jax.experimental import pallas as pl
from jax.experimental.pallas import tpu as pltpu
from jax.experimental.pallas import tpu_sc as plsc

_B, _T, _D = 4, 512, 256
_P, _K = 12, 100
_INVTEMP = 10.0
_NR = _B * _T
_NROWS = _P * _NR
_LANES = 16
_KTOT = _K + 1
_NG = _NROWS // _LANES
_NCORES, _NSUB = 2, 16
_NW = _NCORES * _NSUB
_GPW = _NG // _NW


def _build_gather_cols():
    r_in_step = jnp.arange(_NR, dtype=jnp.int32)
    cols = []
    for s in range(_P):
        step = s + 1
        t2 = _T - step
        kstep = jax.random.fold_in(jax.random.key(42), step)
        neg = jax.random.randint(kstep, (_B, t2, _K), 0, _B * t2)
        negc = (neg // t2) * _T + (neg % t2) + step
        negc = jnp.pad(negc, ((0, 0), (0, _T - t2), (0, 0)))
        posc = jnp.minimum(r_in_step + step, _NR - 1).reshape(_B, _T, 1)
        cols.append(jnp.concatenate([posc, negc], axis=2))
    allc = jnp.stack(cols).reshape(_NG, _LANES, _KTOT)
    lanes = jnp.arange(_LANES, dtype=jnp.int32)[None, :, None]
    allc = ((allc >> 7) * (_LANES * 128) + lanes * 128 + (allc & 127))
    return allc.transpose(0, 2, 1).reshape(-1).astype(jnp.int32)


def _sim_body(ctx_ref, tgt_ref, w_ref, b_ref, s_ref):
    tgt = tgt_ref[...]
    tn = tgt / jnp.maximum(
        jnp.sqrt(jnp.sum(tgt * tgt, axis=1, keepdims=True)), 1e-12)
    q = lax.dot_general(ctx_ref[...], w_ref[0], (((1,), (1,)), ((), ())),
                        preferred_element_type=jnp.float32)
    q = q + b_ref[0, 0][None, :]
    qn = q / jnp.maximum(
        jnp.sqrt(jnp.sum(q * q, axis=1, keepdims=True)), 1e-12)
    for cb in range(_NR // 128):
        s_ref[pl.ds(cb * _NR, _NR), :] = lax.dot_general(
            qn, tn[cb * 128:(cb + 1) * 128, :], (((1,), (1,)), ((), ())),
            preferred_element_type=jnp.float32)


_sim = pl.pallas_call(
    _sim_body,
    grid=(_P,),
    in_specs=[
        pl.BlockSpec((_NR, _D), lambda s: (0, 0)),
        pl.BlockSpec((_NR, _D), lambda s: (0, 0)),
        pl.BlockSpec((1, _D, _D), lambda s: (s, 0, 0)),
        pl.BlockSpec((1, 1, _D), lambda s: (s, 0, 0)),
    ],
    out_specs=pl.BlockSpec(((_NR // 128) * _NR, 128), lambda s: (s, 0)),
    out_shape=jax.ShapeDtypeStruct((_P * (_NR // 128) * _NR, 128),
                                   jnp.float32),
)


_NCB = _NR // 128
_CHUNK = _LANES * 128


def _sc_gather_body(s_hbm, idx_hbm, sum_hbm, pos_hbm,
                    rowbuf, idxbuf, vsum, vpos, sem):
    cid = lax.axis_index("c")
    sid = lax.axis_index("s")
    wid = sid * _NCORES + cid

    def body(g, carry):
        gg = wid * _GPW + g
        grow = gg * _LANES
        step = gg >> 7
        r0 = (gg & 127) * _LANES
        copies = [
            pltpu.async_copy(
                s_hbm.at[pl.ds((step * _NCB + cb) * (_NR * 128) + r0 * 128,
                               _CHUNK)],
                rowbuf.at[pl.ds(cb * _CHUNK, _CHUNK)],
                sem)
            for cb in range(_NCB)
        ]
        pltpu.sync_copy(idx_hbm.at[pl.ds(gg * _KTOT * _LANES, _KTOT * _LANES)],
                        idxbuf)
        for c in copies:
            c.wait()
        v = plsc.load_gather(rowbuf, [idxbuf[pl.ds(0, _LANES)]])
        vpos[:] = v
        acc = jnp.exp(v * _INVTEMP)
        for k in range(1, _KTOT):
            vk = plsc.load_gather(rowbuf, [idxbuf[pl.ds(k * _LANES, _LANES)]])
            acc = acc + jnp.exp(vk * _INVTEMP)
        vsum[:] = acc
        pltpu.sync_copy(vsum, sum_hbm.at[pl.ds(grow, _LANES)])
        pltpu.sync_copy(vpos, pos_hbm.at[pl.ds(grow, _LANES)])
        return carry

    lax.fori_loop(0, _GPW, body, 0)


@functools.cache
def _get_sc_gather():
    return pl.kernel(
        _sc_gather_body,
        mesh=plsc.VectorSubcoreMesh(core_axis_name="c", subcore_axis_name="s"),
        compiler_params=pltpu.CompilerParams(needs_layout_passes=False),
        out_type=[jax.ShapeDtypeStruct((_NROWS,), jnp.float32),
                  jax.ShapeDtypeStruct((_NROWS,), jnp.float32)],
        scratch_types=[
            pltpu.VMEM((_LANES * _NR,), jnp.float32),
            pltpu.VMEM((_KTOT * _LANES,), jnp.int32),
            pltpu.VMEM((_LANES,), jnp.float32),
            pltpu.VMEM((_LANES,), jnp.float32),
            pltpu.SemaphoreType.DMA,
        ],
    )


def _loss_body(sum_ref, pos_ref, out_ref):
    se = sum_ref[...]
    ps = pos_ref[...]
    srow = lax.broadcasted_iota(jnp.int32, (_P, _NR), 0)
    rcol = lax.broadcasted_iota(jnp.int32, (_P, _NR), 1)
    t2 = (_T - 1) - srow
    valid = (rcol % _T) < t2
    wgt = jnp.where(valid, 1.0, 0.0) / (_P * _B * t2.astype(jnp.float32))
    out_ref[...] = jnp.sum((jnp.log(se) - ps * _INVTEMP) * wgt).reshape(1, 1)


_loss = pl.pallas_call(
    _loss_body,
    out_shape=jax.ShapeDtypeStruct((1, 1), jnp.float32),
)


def kernel(context, targets, W, b):
    ctx2 = context.reshape(_NR, _D)
    tgt2 = targets.reshape(_NR, _D)
    b3 = b.reshape(_P, 1, _D)
    idx = _build_gather_cols()
    sim = _sim(ctx2, tgt2, W, b3)
    se, ps = _get_sc_gather()(sim.reshape(-1), idx)
    return _loss(se.reshape(_P, _NR), ps.reshape(_P, _NR)).reshape(1)
